# Initial kernel scaffold; baseline (speedup 1.0000x reference)
import jax
import jax.numpy as jnp
from jax.experimental import pallas as pl


def kernel(x, W1, W2, Wm, eps1, eps2, adj_t):
    raise NotImplementedError("write your pallas kernel here")


# speedup vs baseline: 3.0386x; 3.0386x over previous
import functools

import jax
import jax.numpy as jnp
from jax import lax
from jax.experimental import pallas as pl
from jax.experimental.pallas import tpu as pltpu
from jax.experimental.pallas import tpu_sc as plsc

N_NODES = 10000
NPAD = 10240
N_EDGES = 320000
D = 128

NC = 2
NS = 16
RNG = 2560
SINK = 512
APAD = RNG + SINK
L = 16
K = 128
NBUF = 4
EPT = 20480
NCHUNK = EPT // K
E_PAD = EPT * NS
ZROWS = APAD // NS


def _make_seg_sum(lo0):
    mesh = plsc.VectorSubcoreMesh(
        core_axis_name="c", subcore_axis_name="s", num_cores=NC
    )

    @functools.partial(
        pl.kernel,
        out_type=jax.ShapeDtypeStruct((NC, APAD, D), jnp.float32),
        mesh=mesh,
        scratch_types=[
            pltpu.VMEM((NCHUNK, K), jnp.int32),
            pltpu.VMEM((NCHUNK, K), jnp.int32),
            [pltpu.VMEM((K, D), jnp.float32) for _ in range(NBUF)],
            [pltpu.SemaphoreType.DMA for _ in range(NBUF)],
            pltpu.VMEM_SHARED((APAD, D), jnp.float32),
        ],
    )
    def k(h_hbm, src_hbm, dst_hbm, z_hbm, out_hbm, src_v, dst_v, bufs, sems, acc):
        cid = lax.axis_index("c")
        sid = lax.axis_index("s")
        pltpu.sync_copy(src_hbm.at[sid], src_v)
        pltpu.sync_copy(dst_hbm.at[sid], dst_v)
        pltpu.sync_copy(z_hbm, acc.at[pl.ds(sid * ZROWS, ZROWS)])

        lo = lo0 + cid * RNG

        @pl.loop(0, NCHUNK)
        def _(r):
            for c in range(K // L):
                d = dst_v[r, pl.ds(c * L, L)]
                owned = (d >= lo) & (d < lo + RNG)
                local = jnp.where(owned, d - lo, RNG + (d & (SINK - 1)))
                dst_v[r, pl.ds(c * L, L)] = local

        plsc.subcore_barrier()

        @pl.loop(0, NCHUNK // NBUF)
        def _(i):
            base = i * NBUF
            descs = [
                pltpu.async_copy(h_hbm.at[src_v.at[base + b]], bufs[b], sems[b])
                for b in range(NBUF)
            ]
            for b in range(NBUF):
                descs[b].wait()
                pltpu.sync_copy(bufs[b], acc.at[dst_v.at[base + b]], add=True)

        plsc.subcore_barrier()
        pltpu.sync_copy(
            acc.at[pl.ds(sid * ZROWS, ZROWS)],
            out_hbm.at[cid, pl.ds(sid * ZROWS, ZROWS)],
        )

    return k


_seg_sum_lo = _make_seg_sum(0)
_seg_sum_hi = _make_seg_sum(2 * RNG)


def _mlp1_body(pa_ref, pb_ref, h_ref, w_ref, eps_ref, o_ref):
    i = pl.program_id(0)
    s = 1.0 + eps_ref[0]
    p = jnp.where(i < 10, pa_ref[0], pb_ref[0])
    a = p + s * h_ref[...]
    o_ref[...] = jnp.maximum(
        jnp.dot(a, w_ref[...], preferred_element_type=jnp.float32), 0.0
    )


def _mlp2_body(pa_ref, pb_ref, x1_ref, w2_ref, wm_ref, eps_ref, o_ref):
    i = pl.program_id(0)
    s = 1.0 + eps_ref[0]
    x1 = x1_ref[...]
    p = jnp.where(i < 10, pa_ref[0], pb_ref[0])
    a = p + s * x1
    t = jnp.maximum(
        jnp.dot(a, w2_ref[...], preferred_element_type=jnp.float32), 0.0
    )
    o_ref[...] = (jnp.dot(x1, wm_ref[...], preferred_element_type=jnp.float32) + t) * 0.5


_R = 512
_G = NPAD // _R
_BS = RNG // _R


def _p_spec():
    return pl.BlockSpec(
        (1, _R, D), lambda i: ((i % (2 * _BS)) // _BS, i % _BS, 0)
    )


def _mlp1(pa, pb, h, w1t, eps1):
    return pl.pallas_call(
        _mlp1_body,
        grid=(_G,),
        in_specs=[
            _p_spec(),
            _p_spec(),
            pl.BlockSpec((_R, D), lambda i: (i, 0)),
            pl.BlockSpec((D, D), lambda i: (0, 0)),
            pl.BlockSpec(memory_space=pltpu.SMEM),
        ],
        out_specs=pl.BlockSpec((_R, D), lambda i: (i, 0)),
        out_shape=jax.ShapeDtypeStruct((NPAD, D), jnp.float32),
    )(pa, pb, h, w1t, eps1)


def _mlp2(pa, pb, x1, w2t, wmt, eps2):
    return pl.pallas_call(
        _mlp2_body,
        grid=(_G,),
        in_specs=[
            _p_spec(),
            _p_spec(),
            pl.BlockSpec((_R, D), lambda i: (i, 0)),
            pl.BlockSpec((D, D), lambda i: (0, 0)),
            pl.BlockSpec((D, D), lambda i: (0, 0)),
            pl.BlockSpec(memory_space=pltpu.SMEM),
        ],
        out_specs=pl.BlockSpec((_R, D), lambda i: (i, 0)),
        out_shape=jax.ShapeDtypeStruct((NPAD, D), jnp.float32),
    )(pa, pb, x1, w2t, wmt, eps2)


def kernel(x, W1, W2, Wm, eps1, eps2, adj_t):
    adj = adj_t.astype(jnp.int32)
    pad = E_PAD - N_EDGES
    pidx = jnp.arange(pad, dtype=jnp.int32)
    pad_src = pidx % N_NODES
    pad_dst = 4 * RNG + (pidx & (SINK - 1))
    src_g = jnp.concatenate([adj[0], pad_src]).reshape(NS, NCHUNK, K)
    dst_g = jnp.concatenate([adj[1], pad_dst]).reshape(NS, NCHUNK, K)
    zeros = jnp.zeros((ZROWS, D), jnp.float32)
    x_p = jnp.pad(x, ((0, NPAD - N_NODES), (0, 0)))

    p1a = _seg_sum_lo(x_p, src_g, dst_g, zeros)
    p1b = _seg_sum_hi(x_p, src_g, dst_g, zeros)
    x1 = _mlp1(p1a, p1b, x_p, W1.T, eps1.reshape(1))
    p2a = _seg_sum_lo(x1, src_g, dst_g, zeros)
    p2b = _seg_sum_hi(x1, src_g, dst_g, zeros)
    out = _mlp2(p2a, p2b, x1, W2.T, Wm.T, eps2.reshape(1))
    return out[:N_NODES]


# speedup vs baseline: 4.6481x; 1.5297x over previous
import functools

import jax
import jax.numpy as jnp
from jax import lax
from jax.experimental import pallas as pl
from jax.experimental.pallas import tpu as pltpu
from jax.experimental.pallas import tpu_sc as plsc

N_NODES = 10000
NPAD = 10240
N_EDGES = 320000
D = 128

NC = 2
NS = 16
RNG = 2560
SINK = 512
APAD = RNG + SINK
L = 16
K = 128
NBUF = 4
EPT = 20480
NCHUNK = EPT // K
E_PAD = EPT * NS
ZROWS = APAD // NS


def _make_seg_sum(lo0):
    mesh = plsc.VectorSubcoreMesh(
        core_axis_name="c", subcore_axis_name="s", num_cores=NC
    )

    @functools.partial(
        pl.kernel,
        out_type=jax.ShapeDtypeStruct((NC, APAD, D), jnp.float32),
        mesh=mesh,
        scratch_types=[
            pltpu.VMEM((NCHUNK, K), jnp.int32),
            pltpu.VMEM((NCHUNK, K), jnp.int32),
            [pltpu.VMEM((K, D), jnp.float32) for _ in range(NBUF)],
            [pltpu.SemaphoreType.DMA for _ in range(NBUF)],
            pltpu.VMEM_SHARED((APAD, D), jnp.float32),
        ],
    )
    def k(h_hbm, src_hbm, dst_hbm, z_hbm, out_hbm, src_v, dst_v, bufs, sems, acc):
        cid = lax.axis_index("c")
        sid = lax.axis_index("s")
        pltpu.sync_copy(src_hbm.at[sid], src_v)
        pltpu.sync_copy(dst_hbm.at[sid], dst_v)
        pltpu.sync_copy(z_hbm, acc.at[pl.ds(sid * ZROWS, ZROWS)])

        lo = lo0 + cid * RNG

        @pl.loop(0, NCHUNK)
        def _(r):
            for c in range(K // L):
                d = dst_v[r, pl.ds(c * L, L)]
                owned = (d >= lo) & (d < lo + RNG)
                local = jnp.where(owned, d - lo, RNG + (d & (SINK - 1)))
                dst_v[r, pl.ds(c * L, L)] = local

        plsc.subcore_barrier()

        for b in range(NBUF):
            pltpu.async_copy(h_hbm.at[src_v.at[b]], bufs[b], sems[b])

        @pl.loop(0, NCHUNK // NBUF)
        def _(i):
            base = i * NBUF
            for b in range(NBUF):
                pltpu.make_async_copy(
                    h_hbm.at[src_v.at[base + b]], bufs[b], sems[b]
                ).wait()
                pltpu.sync_copy(bufs[b], acc.at[dst_v.at[base + b]], add=True)

                @pl.when(base + b + NBUF < NCHUNK)
                def _():
                    pltpu.async_copy(
                        h_hbm.at[src_v.at[base + b + NBUF]], bufs[b], sems[b]
                    )

        plsc.subcore_barrier()
        pltpu.sync_copy(
            acc.at[pl.ds(sid * ZROWS, ZROWS)],
            out_hbm.at[cid, pl.ds(sid * ZROWS, ZROWS)],
        )

    return k


_seg_sum_lo = _make_seg_sum(0)
_seg_sum_hi = _make_seg_sum(2 * RNG)


def _mlp1_body(pa_ref, pb_ref, h_ref, w_ref, eps_ref, o_ref):
    i = pl.program_id(0)
    s = 1.0 + eps_ref[0]
    p = jnp.where(i < 10, pa_ref[0], pb_ref[0])
    a = p + s * h_ref[...]
    o_ref[...] = jnp.maximum(
        jnp.dot(a, w_ref[...], preferred_element_type=jnp.float32), 0.0
    )


def _mlp2_body(pa_ref, pb_ref, x1_ref, w2_ref, wm_ref, eps_ref, o_ref):
    i = pl.program_id(0)
    s = 1.0 + eps_ref[0]
    x1 = x1_ref[...]
    p = jnp.where(i < 10, pa_ref[0], pb_ref[0])
    a = p + s * x1
    t = jnp.maximum(
        jnp.dot(a, w2_ref[...], preferred_element_type=jnp.float32), 0.0
    )
    o_ref[...] = (jnp.dot(x1, wm_ref[...], preferred_element_type=jnp.float32) + t) * 0.5


_R = 512
_G = NPAD // _R
_BS = RNG // _R


def _p_spec():
    return pl.BlockSpec(
        (1, _R, D), lambda i: ((i % (2 * _BS)) // _BS, i % _BS, 0)
    )


def _mlp1(pa, pb, h, w1t, eps1):
    return pl.pallas_call(
        _mlp1_body,
        grid=(_G,),
        in_specs=[
            _p_spec(),
            _p_spec(),
            pl.BlockSpec((_R, D), lambda i: (i, 0)),
            pl.BlockSpec((D, D), lambda i: (0, 0)),
            pl.BlockSpec(memory_space=pltpu.SMEM),
        ],
        out_specs=pl.BlockSpec((_R, D), lambda i: (i, 0)),
        out_shape=jax.ShapeDtypeStruct((NPAD, D), jnp.float32),
    )(pa, pb, h, w1t, eps1)


def _mlp2(pa, pb, x1, w2t, wmt, eps2):
    return pl.pallas_call(
        _mlp2_body,
        grid=(_G,),
        in_specs=[
            _p_spec(),
            _p_spec(),
            pl.BlockSpec((_R, D), lambda i: (i, 0)),
            pl.BlockSpec((D, D), lambda i: (0, 0)),
            pl.BlockSpec((D, D), lambda i: (0, 0)),
            pl.BlockSpec(memory_space=pltpu.SMEM),
        ],
        out_specs=pl.BlockSpec((_R, D), lambda i: (i, 0)),
        out_shape=jax.ShapeDtypeStruct((NPAD, D), jnp.float32),
    )(pa, pb, x1, w2t, wmt, eps2)


def kernel(x, W1, W2, Wm, eps1, eps2, adj_t):
    adj = adj_t.astype(jnp.int32)
    pad = E_PAD - N_EDGES
    pidx = jnp.arange(pad, dtype=jnp.int32)
    pad_src = pidx % N_NODES
    pad_dst = 4 * RNG + (pidx & (SINK - 1))
    src_g = jnp.concatenate([adj[0], pad_src]).reshape(NS, NCHUNK, K)
    dst_g = jnp.concatenate([adj[1], pad_dst]).reshape(NS, NCHUNK, K)
    zeros = jnp.zeros((ZROWS, D), jnp.float32)
    x_p = jnp.pad(x, ((0, NPAD - N_NODES), (0, 0)))

    p1a = _seg_sum_lo(x_p, src_g, dst_g, zeros)
    p1b = _seg_sum_hi(x_p, src_g, dst_g, zeros)
    x1 = _mlp1(p1a, p1b, x_p, W1.T, eps1.reshape(1))
    p2a = _seg_sum_lo(x1, src_g, dst_g, zeros)
    p2b = _seg_sum_hi(x1, src_g, dst_g, zeros)
    out = _mlp2(p2a, p2b, x1, W2.T, Wm.T, eps2.reshape(1))
    return out[:N_NODES]


# speedup vs baseline: 4.7170x; 1.0148x over previous
import functools

import jax
import jax.numpy as jnp
from jax import lax
from jax.experimental import pallas as pl
from jax.experimental.pallas import tpu as pltpu
from jax.experimental.pallas import tpu_sc as plsc

N_NODES = 10000
NPAD = 10240
N_EDGES = 320000
D = 128

NC = 2
NS = 16
RNG = 2560
SINK = 512
APAD = RNG + SINK
L = 16
K = 128
NBUF = 4
EPT = 20480
NCHUNK = EPT // K
E_PAD = EPT * NS
ZROWS = APAD // NS


def _make_seg_sum(lo0):
    mesh = plsc.VectorSubcoreMesh(
        core_axis_name="c", subcore_axis_name="s", num_cores=NC
    )

    @functools.partial(
        pl.kernel,
        out_type=jax.ShapeDtypeStruct((NC, APAD, D), jnp.float32),
        mesh=mesh,
        scratch_types=[
            pltpu.VMEM((NCHUNK, K), jnp.int32),
            pltpu.VMEM((NCHUNK, K), jnp.int32),
            [pltpu.VMEM((K, D), jnp.float32) for _ in range(NBUF)],
            [pltpu.SemaphoreType.DMA for _ in range(NBUF)],
            pltpu.VMEM_SHARED((APAD, D), jnp.float32),
        ],
    )
    def k(h_hbm, src_hbm, dst_hbm, z_hbm, out_hbm, src_v, dst_v, bufs, sems, acc):
        cid = lax.axis_index("c")
        sid = lax.axis_index("s")
        c1 = pltpu.async_copy(src_hbm.at[sid], src_v, sems[0])
        c2 = pltpu.async_copy(dst_hbm.at[sid], dst_v, sems[1])
        c3 = pltpu.async_copy(z_hbm, acc.at[pl.ds(sid * ZROWS, ZROWS)], sems[2])
        c1.wait()
        c2.wait()
        c3.wait()

        lo = lo0 + cid * RNG
        plsc.subcore_barrier()

        for b in range(NBUF):
            pltpu.async_copy(h_hbm.at[src_v.at[b]], bufs[b], sems[b])

        @pl.loop(0, NCHUNK // NBUF)
        def _(i):
            base = i * NBUF
            for b in range(NBUF):
                for c in range(K // L):
                    d = dst_v[base + b, pl.ds(c * L, L)]
                    owned = (d >= lo) & (d < lo + RNG)
                    local = jnp.where(owned, d - lo, RNG + (d & (SINK - 1)))
                    dst_v[base + b, pl.ds(c * L, L)] = local
                pltpu.make_async_copy(
                    h_hbm.at[src_v.at[base + b]], bufs[b], sems[b]
                ).wait()
                pltpu.sync_copy(bufs[b], acc.at[dst_v.at[base + b]], add=True)

                @pl.when(base + b + NBUF < NCHUNK)
                def _():
                    pltpu.async_copy(
                        h_hbm.at[src_v.at[base + b + NBUF]], bufs[b], sems[b]
                    )

        plsc.subcore_barrier()
        pltpu.sync_copy(
            acc.at[pl.ds(sid * ZROWS, ZROWS)],
            out_hbm.at[cid, pl.ds(sid * ZROWS, ZROWS)],
        )

    return k


_seg_sum_lo = _make_seg_sum(0)
_seg_sum_hi = _make_seg_sum(2 * RNG)


def _mlp1_body(pa_ref, pb_ref, h_ref, w_ref, eps_ref, o_ref):
    i = pl.program_id(0)
    s = 1.0 + eps_ref[0]
    p = jnp.where(i < 10, pa_ref[0], pb_ref[0])
    a = p + s * h_ref[...]
    o_ref[...] = jnp.maximum(
        jnp.dot(a, w_ref[...], preferred_element_type=jnp.float32), 0.0
    )


def _mlp2_body(pa_ref, pb_ref, x1_ref, w2_ref, wm_ref, eps_ref, o_ref):
    i = pl.program_id(0)
    s = 1.0 + eps_ref[0]
    x1 = x1_ref[...]
    p = jnp.where(i < 10, pa_ref[0], pb_ref[0])
    a = p + s * x1
    t = jnp.maximum(
        jnp.dot(a, w2_ref[...], preferred_element_type=jnp.float32), 0.0
    )
    o_ref[...] = (jnp.dot(x1, wm_ref[...], preferred_element_type=jnp.float32) + t) * 0.5


_R = 512
_G = NPAD // _R
_BS = RNG // _R


def _p_spec():
    return pl.BlockSpec(
        (1, _R, D), lambda i: ((i % (2 * _BS)) // _BS, i % _BS, 0)
    )


def _mlp1(pa, pb, h, w1t, eps1):
    return pl.pallas_call(
        _mlp1_body,
        grid=(_G,),
        in_specs=[
            _p_spec(),
            _p_spec(),
            pl.BlockSpec((_R, D), lambda i: (i, 0)),
            pl.BlockSpec((D, D), lambda i: (0, 0)),
            pl.BlockSpec(memory_space=pltpu.SMEM),
        ],
        out_specs=pl.BlockSpec((_R, D), lambda i: (i, 0)),
        out_shape=jax.ShapeDtypeStruct((NPAD, D), jnp.float32),
    )(pa, pb, h, w1t, eps1)


def _mlp2(pa, pb, x1, w2t, wmt, eps2):
    return pl.pallas_call(
        _mlp2_body,
        grid=(_G,),
        in_specs=[
            _p_spec(),
            _p_spec(),
            pl.BlockSpec((_R, D), lambda i: (i, 0)),
            pl.BlockSpec((D, D), lambda i: (0, 0)),
            pl.BlockSpec((D, D), lambda i: (0, 0)),
            pl.BlockSpec(memory_space=pltpu.SMEM),
        ],
        out_specs=pl.BlockSpec((_R, D), lambda i: (i, 0)),
        out_shape=jax.ShapeDtypeStruct((NPAD, D), jnp.float32),
    )(pa, pb, x1, w2t, wmt, eps2)


def kernel(x, W1, W2, Wm, eps1, eps2, adj_t):
    adj = adj_t.astype(jnp.int32)
    pad = E_PAD - N_EDGES
    pidx = jnp.arange(pad, dtype=jnp.int32)
    pad_src = pidx % N_NODES
    pad_dst = 4 * RNG + (pidx & (SINK - 1))
    src_g = jnp.concatenate([adj[0], pad_src]).reshape(NS, NCHUNK, K)
    dst_g = jnp.concatenate([adj[1], pad_dst]).reshape(NS, NCHUNK, K)
    zeros = jnp.zeros((ZROWS, D), jnp.float32)
    x_p = jnp.pad(x, ((0, NPAD - N_NODES), (0, 0)))

    p1a = _seg_sum_lo(x_p, src_g, dst_g, zeros)
    p1b = _seg_sum_hi(x_p, src_g, dst_g, zeros)
    x1 = _mlp1(p1a, p1b, x_p, W1.T, eps1.reshape(1))
    p2a = _seg_sum_lo(x1, src_g, dst_g, zeros)
    p2b = _seg_sum_hi(x1, src_g, dst_g, zeros)
    out = _mlp2(p2a, p2b, x1, W2.T, Wm.T, eps2.reshape(1))
    return out[:N_NODES]
